# baseline (speedup 1.0000x reference)
import functools

import jax
import jax.numpy as jnp
from jax import lax
from jax.experimental import pallas as pl
from jax.experimental.pallas import tpu as pltpu
from jax.experimental.pallas import tpu_sc as plsc

EMBED_DIM = 10
HIDDEN = 128
BATCH = 16384
DV = 100000
AV = 1000000

NUM_CORES = 2
NUM_SUBCORES = 16
NUM_WORKERS = NUM_CORES * NUM_SUBCORES
B_PER_W = BATCH // NUM_WORKERS
CHUNK = 128
NCHUNK = B_PER_W // CHUNK
ROW_BLOCKS = BATCH // CHUNK
LANES = 16


def _detile_body(dt_ref, at_ref, *rest):
  outs, sem = rest[:-1], rest[-1]
  copies = []
  for j in range(EMBED_DIM):
    copies.append(pltpu.async_copy(dt_ref.at[j], outs[j], sem))
    copies.append(pltpu.async_copy(at_ref.at[j], outs[EMBED_DIM + j], sem))
  for c in copies:
    c.wait()


def _detile(dt, at):
  return pl.pallas_call(
      _detile_body,
      out_shape=(
          [jax.ShapeDtypeStruct((DV,), jnp.float32)] * EMBED_DIM
          + [jax.ShapeDtypeStruct((AV,), jnp.float32)] * EMBED_DIM),
      in_specs=[
          pl.BlockSpec(memory_space=pl.ANY),
          pl.BlockSpec(memory_space=pl.ANY),
      ],
      out_specs=[pl.BlockSpec(memory_space=pl.ANY)] * (2 * EMBED_DIM),
      scratch_shapes=[pltpu.SemaphoreType.DMA],
  )(dt, at)


def _gather_body(did_hbm, aid_hbm, *rest):
  dtabs = rest[:EMBED_DIM]
  atabs = rest[EMBED_DIM:2 * EMBED_DIM]
  outd_hbm, outa_hbm = rest[2 * EMBED_DIM:2 * EMBED_DIM + 2]
  (idxd_v, idxa_v, k0d_v, k0a_v, staged_v, stagea_v, sem) = \
      rest[2 * EMBED_DIM + 2:]
  wid = lax.axis_index("s") * NUM_CORES + lax.axis_index("c")
  base = wid * NCHUNK
  pltpu.sync_copy(did_hbm.at[pl.ds(base, NCHUNK)], idxd_v)
  pltpu.sync_copy(aid_hbm.at[pl.ds(base, NCHUNK)], idxa_v)
  for c in range(NCHUNK):
    for g in range(CHUNK // LANES):
      sl = pl.ds(g * LANES, LANES)
      k0d_v[sl] = lax.shift_right_logical(idxd_v.at[c][sl], 3)
      k0a_v[sl] = lax.shift_right_logical(idxa_v.at[c][sl], 3)
    copies = []
    for j in range(EMBED_DIM):
      copies.append(pltpu.async_copy(
          dtabs[j].at[k0d_v], staged_v.at[j], sem))
      copies.append(pltpu.async_copy(
          atabs[j].at[k0a_v], stagea_v.at[j], sem))
    for cp in copies:
      cp.wait()
    out0 = (base + c) * CHUNK
    copies = []
    for j in range(EMBED_DIM):
      copies.append(pltpu.async_copy(
          staged_v.at[j], outd_hbm.at[j].at[pl.ds(out0, CHUNK)], sem))
      copies.append(pltpu.async_copy(
          stagea_v.at[j], outa_hbm.at[j].at[pl.ds(out0, CHUNK)], sem))
    for cp in copies:
      cp.wait()


@functools.cache
def _sc_gather():
  return functools.partial(
      pl.kernel,
      out_type=[
          jax.ShapeDtypeStruct((EMBED_DIM, BATCH, 8), jnp.float32),
          jax.ShapeDtypeStruct((EMBED_DIM, BATCH, 8), jnp.float32),
      ],
      mesh=plsc.VectorSubcoreMesh(core_axis_name="c", subcore_axis_name="s",
                                  num_cores=NUM_CORES,
                                  num_subcores=NUM_SUBCORES),
      scratch_types=[
          pltpu.VMEM((NCHUNK, CHUNK), jnp.int32),
          pltpu.VMEM((NCHUNK, CHUNK), jnp.int32),
          pltpu.VMEM((CHUNK,), jnp.int32),
          pltpu.VMEM((CHUNK,), jnp.int32),
          pltpu.VMEM((EMBED_DIM, CHUNK, 8), jnp.float32),
          pltpu.VMEM((EMBED_DIM, CHUNK, 8), jnp.float32),
          pltpu.SemaphoreType.DMA,
      ],
      compiler_params=pltpu.CompilerParams(use_tc_tiling_on_sc=False),
  )(_gather_body)


def _mlp_body(bufd_ref, bufa_ref, did_ref, aid_ref, w1d_ref, w1a_ref,
              b1_ref, w2_ref, b2_ref, o_ref):
  blk = did_ref.shape[0]
  oct8 = jax.lax.broadcasted_iota(jnp.int32, (blk, 8), 1)
  dm = (oct8 == (did_ref[...] & 7).reshape(blk, 1)).astype(jnp.float32)
  am = (oct8 == (aid_ref[...] & 7).reshape(blk, 1)).astype(jnp.float32)
  xd = jnp.concatenate(
      [jnp.sum(bufd_ref[j] * dm, axis=1, keepdims=True)
       for j in range(EMBED_DIM)], axis=1)
  xa = jnp.concatenate(
      [jnp.sum(bufa_ref[j] * am, axis=1, keepdims=True)
       for j in range(EMBED_DIM)], axis=1)
  h = jnp.dot(xd, w1d_ref[...], preferred_element_type=jnp.float32)
  h = h + jnp.dot(xa, w1a_ref[...], preferred_element_type=jnp.float32)
  h = jnp.maximum(h + b1_ref[...], 0.0)
  o_ref[...] = jnp.sum(h * w2_ref[...], axis=1) + b2_ref[0, 0]


def _mlp(bufd, bufa, did, aid, w1d, w1a, b1r, w2r, b2r):
  blk = 2048
  grid = BATCH // blk
  return pl.pallas_call(
      _mlp_body,
      out_shape=jax.ShapeDtypeStruct((BATCH,), jnp.float32),
      grid=(grid,),
      in_specs=[
          pl.BlockSpec((EMBED_DIM, blk, 8), lambda i: (0, i, 0)),
          pl.BlockSpec((EMBED_DIM, blk, 8), lambda i: (0, i, 0)),
          pl.BlockSpec((blk,), lambda i: (i,)),
          pl.BlockSpec((blk,), lambda i: (i,)),
          pl.BlockSpec((EMBED_DIM, HIDDEN), lambda i: (0, 0)),
          pl.BlockSpec((EMBED_DIM, HIDDEN), lambda i: (0, 0)),
          pl.BlockSpec((1, HIDDEN), lambda i: (0, 0)),
          pl.BlockSpec((1, HIDDEN), lambda i: (0, 0)),
          pl.BlockSpec((1, 1), lambda i: (0, 0)),
      ],
      out_specs=pl.BlockSpec((blk,), lambda i: (i,)),
  )(bufd, bufa, did, aid, w1d, w1a, b1r, w2r, b2r)


def kernel(domain_id, author_id, domain_table, author_table, W1, b1, W2, b2):
  did = domain_id.astype(jnp.int32)
  aid = author_id.astype(jnp.int32)
  cols = _detile(domain_table.T, author_table.T)
  dtabs = [c.reshape(DV // 8, 8) for c in cols[:EMBED_DIM]]
  atabs = [c.reshape(AV // 8, 8) for c in cols[EMBED_DIM:]]
  did2 = did.reshape(ROW_BLOCKS, CHUNK)
  aid2 = aid.reshape(ROW_BLOCKS, CHUNK)
  bufd, bufa = _sc_gather()(did2, aid2, *dtabs, *atabs)
  w1d = W1[:EMBED_DIM]
  w1a = W1[EMBED_DIM:]
  b1r = b1.reshape(1, HIDDEN)
  w2r = W2.reshape(1, HIDDEN)
  b2r = b2.reshape(1, 1)
  return _mlp(bufd, bufa, did, aid, w1d, w1a, b1r, w2r, b2r)


# speedup vs baseline: 5.0348x; 5.0348x over previous
import functools

import jax
import jax.numpy as jnp
from jax import lax
from jax.experimental import pallas as pl
from jax.experimental.pallas import tpu as pltpu
from jax.experimental.pallas import tpu_sc as plsc

EMBED_DIM = 10
HIDDEN = 128
BATCH = 16384
DV = 100000
AV = 1000000

NUM_CORES = 2
NUM_SUBCORES = 16
NUM_WORKERS = NUM_CORES * NUM_SUBCORES
B_PER_W = BATCH // NUM_WORKERS
CHUNK = 128
NCHUNK = B_PER_W // CHUNK
ROW_BLOCKS = BATCH // CHUNK
LANES = 16

TILE_ROWS = 16
LANE_BLK = 8192
TPB = LANE_BLK // 128
ROWS_PB = TPB * TILE_ROWS


def _tr_body(src_ref, o_ref):
  x = src_ref[...]
  x16 = jnp.concatenate([x, x[:6]], axis=0)
  for t in range(TPB):
    o_ref[pl.ds(t * TILE_ROWS, TILE_ROWS), :] = x16[:, t * 128:(t + 1) * 128]


def _detile(tab_t, vocab):
  nb = -(-vocab // LANE_BLK)
  return pl.pallas_call(
      _tr_body,
      out_shape=jax.ShapeDtypeStruct((nb * ROWS_PB, 128), jnp.float32),
      grid=(nb,),
      in_specs=[pl.BlockSpec((EMBED_DIM, LANE_BLK), lambda c: (0, c))],
      out_specs=pl.BlockSpec((ROWS_PB, 128), lambda c: (c, 0)),
  )(tab_t)


def _gather_body(did_hbm, aid_hbm, dtab, atab, outd_hbm, outa_hbm, *scr):
  idxd_v, idxa_v = scr[0], scr[1]
  kds = scr[2:2 + EMBED_DIM]
  kas = scr[2 + EMBED_DIM:2 + 2 * EMBED_DIM]
  staged_v, stagea_v, sem = scr[-3], scr[-2], scr[-1]
  wid = lax.axis_index("s") * NUM_CORES + lax.axis_index("c")
  base = wid * NCHUNK
  pltpu.sync_copy(did_hbm.at[pl.ds(base, NCHUNK)], idxd_v)
  pltpu.sync_copy(aid_hbm.at[pl.ds(base, NCHUNK)], idxa_v)
  for c in range(NCHUNK):
    for g in range(CHUNK // LANES):
      sl = pl.ds(g * LANES, LANES)
      vd = idxd_v.at[c][sl]
      va = idxa_v.at[c][sl]
      bd = lax.shift_right_logical(vd, 3) + lax.shift_right_logical(vd, 7) * 240
      ba = lax.shift_right_logical(va, 3) + lax.shift_right_logical(va, 7) * 240
      for j in range(EMBED_DIM):
        kds[j][sl] = bd + j * 16
        kas[j][sl] = ba + j * 16
    copies = []
    for j in range(EMBED_DIM):
      copies.append(pltpu.async_copy(dtab.at[kds[j]], staged_v.at[j], sem))
      copies.append(pltpu.async_copy(atab.at[kas[j]], stagea_v.at[j], sem))
    for cp in copies:
      cp.wait()
    out0 = (base + c) * CHUNK
    copies = []
    for j in range(EMBED_DIM):
      copies.append(pltpu.async_copy(
          staged_v.at[j], outd_hbm.at[j].at[pl.ds(out0, CHUNK)], sem))
      copies.append(pltpu.async_copy(
          stagea_v.at[j], outa_hbm.at[j].at[pl.ds(out0, CHUNK)], sem))
    for cp in copies:
      cp.wait()


@functools.cache
def _sc_gather():
  return functools.partial(
      pl.kernel,
      out_type=[
          jax.ShapeDtypeStruct((EMBED_DIM, BATCH, 8), jnp.float32),
          jax.ShapeDtypeStruct((EMBED_DIM, BATCH, 8), jnp.float32),
      ],
      mesh=plsc.VectorSubcoreMesh(core_axis_name="c", subcore_axis_name="s",
                                  num_cores=NUM_CORES,
                                  num_subcores=NUM_SUBCORES),
      scratch_types=(
          [pltpu.VMEM((NCHUNK, CHUNK), jnp.int32)] * 2
          + [pltpu.VMEM((CHUNK,), jnp.int32)] * (2 * EMBED_DIM)
          + [pltpu.VMEM((EMBED_DIM, CHUNK, 8), jnp.float32)] * 2
          + [pltpu.SemaphoreType.DMA]),
      compiler_params=pltpu.CompilerParams(use_tc_tiling_on_sc=False),
  )(_gather_body)


def _mlp_body(bufd_ref, bufa_ref, did_ref, aid_ref, w1d_ref, w1a_ref,
              b1_ref, w2_ref, b2_ref, o_ref):
  blk = did_ref.shape[0]
  oct8 = jax.lax.broadcasted_iota(jnp.int32, (blk, 8), 1)
  dm = (oct8 == (did_ref[...] & 7).reshape(blk, 1)).astype(jnp.float32)
  am = (oct8 == (aid_ref[...] & 7).reshape(blk, 1)).astype(jnp.float32)
  xd = jnp.concatenate(
      [jnp.sum(bufd_ref[j] * dm, axis=1, keepdims=True)
       for j in range(EMBED_DIM)], axis=1)
  xa = jnp.concatenate(
      [jnp.sum(bufa_ref[j] * am, axis=1, keepdims=True)
       for j in range(EMBED_DIM)], axis=1)
  h = jnp.dot(xd, w1d_ref[...], preferred_element_type=jnp.float32)
  h = h + jnp.dot(xa, w1a_ref[...], preferred_element_type=jnp.float32)
  h = jnp.maximum(h + b1_ref[...], 0.0)
  o_ref[...] = jnp.sum(h * w2_ref[...], axis=1) + b2_ref[0, 0]


def _mlp(bufd, bufa, did, aid, w1d, w1a, b1r, w2r, b2r):
  blk = 2048
  grid = BATCH // blk
  return pl.pallas_call(
      _mlp_body,
      out_shape=jax.ShapeDtypeStruct((BATCH,), jnp.float32),
      grid=(grid,),
      in_specs=[
          pl.BlockSpec((EMBED_DIM, blk, 8), lambda i: (0, i, 0)),
          pl.BlockSpec((EMBED_DIM, blk, 8), lambda i: (0, i, 0)),
          pl.BlockSpec((blk,), lambda i: (i,)),
          pl.BlockSpec((blk,), lambda i: (i,)),
          pl.BlockSpec((EMBED_DIM, HIDDEN), lambda i: (0, 0)),
          pl.BlockSpec((EMBED_DIM, HIDDEN), lambda i: (0, 0)),
          pl.BlockSpec((1, HIDDEN), lambda i: (0, 0)),
          pl.BlockSpec((1, HIDDEN), lambda i: (0, 0)),
          pl.BlockSpec((1, 1), lambda i: (0, 0)),
      ],
      out_specs=pl.BlockSpec((blk,), lambda i: (i,)),
  )(bufd, bufa, did, aid, w1d, w1a, b1r, w2r, b2r)


def kernel(domain_id, author_id, domain_table, author_table, W1, b1, W2, b2):
  did = domain_id.astype(jnp.int32)
  aid = author_id.astype(jnp.int32)
  dtab = _detile(domain_table.T, DV).reshape(-1, 8)
  atab = _detile(author_table.T, AV).reshape(-1, 8)
  did2 = did.reshape(ROW_BLOCKS, CHUNK)
  aid2 = aid.reshape(ROW_BLOCKS, CHUNK)
  bufd, bufa = _sc_gather()(did2, aid2, dtab, atab)
  w1d = W1[:EMBED_DIM]
  w1a = W1[EMBED_DIM:]
  b1r = b1.reshape(1, HIDDEN)
  w2r = W2.reshape(1, HIDDEN)
  b2r = b2.reshape(1, 1)
  return _mlp(bufd, bufa, did, aid, w1d, w1a, b1r, w2r, b2r)


# speedup vs baseline: 5.9600x; 1.1838x over previous
import functools

import jax
import jax.numpy as jnp
from jax import lax
from jax.experimental import pallas as pl
from jax.experimental.pallas import tpu as pltpu
from jax.experimental.pallas import tpu_sc as plsc

EMBED_DIM = 10
HIDDEN = 128
BATCH = 16384
DV = 100000
AV = 1000000

NUM_CORES = 2
NUM_SUBCORES = 16
NUM_WORKERS = NUM_CORES * NUM_SUBCORES
B_PER_W = BATCH // NUM_WORKERS
CHUNK = 128
NCHUNK = B_PER_W // CHUNK
ROW_BLOCKS = BATCH // CHUNK
LANES = 16

TILE_ROWS = 16
LANE_BLK = 32768


def _tr_body(tpb, src_ref, o_ref):
  x = src_ref[...]
  x16 = jnp.concatenate([x, x[:6]], axis=0)
  for t in range(tpb):
    o_ref[pl.ds(t * TILE_ROWS, TILE_ROWS), :] = x16[:, t * 128:(t + 1) * 128]


def _detile(tab_t, vocab):
  nb = -(-vocab // LANE_BLK)
  tpb = LANE_BLK // 128
  rows_pb = tpb * TILE_ROWS
  return pl.pallas_call(
      functools.partial(_tr_body, tpb),
      out_shape=jax.ShapeDtypeStruct((nb * rows_pb, 128), jnp.float32),
      grid=(nb,),
      in_specs=[pl.BlockSpec((EMBED_DIM, LANE_BLK), lambda c: (0, c))],
      out_specs=pl.BlockSpec((rows_pb, 128), lambda c: (c, 0)),
  )(tab_t)


def _gather_body(did_hbm, aid_hbm, dtab, atab, outd_hbm, outa_hbm, *scr):
  idxd_v, idxa_v = scr[0], scr[1]
  kds = scr[2:2 + EMBED_DIM]
  kas = scr[2 + EMBED_DIM:2 + 2 * EMBED_DIM]
  staged_v, stagea_v, sem = scr[-3], scr[-2], scr[-1]
  wid = lax.axis_index("s") * NUM_CORES + lax.axis_index("c")
  base = wid * NCHUNK
  pltpu.sync_copy(did_hbm.at[pl.ds(base, NCHUNK)], idxd_v)
  pltpu.sync_copy(aid_hbm.at[pl.ds(base, NCHUNK)], idxa_v)
  for c in range(NCHUNK):
    for g in range(CHUNK // LANES):
      sl = pl.ds(g * LANES, LANES)
      vd = idxd_v.at[c][sl]
      va = idxa_v.at[c][sl]
      bd = lax.shift_right_logical(vd, 3) + lax.shift_right_logical(vd, 7) * 240
      ba = lax.shift_right_logical(va, 3) + lax.shift_right_logical(va, 7) * 240
      for j in range(EMBED_DIM):
        kds[j][sl] = bd + j * 16
        kas[j][sl] = ba + j * 16
    copies = []
    for j in range(EMBED_DIM):
      copies.append(pltpu.async_copy(dtab.at[kds[j]], staged_v.at[j], sem))
      copies.append(pltpu.async_copy(atab.at[kas[j]], stagea_v.at[j], sem))
    for cp in copies:
      cp.wait()
    out0 = (base + c) * CHUNK
    copies = []
    for j in range(EMBED_DIM):
      copies.append(pltpu.async_copy(
          staged_v.at[j], outd_hbm.at[j].at[pl.ds(out0, CHUNK)], sem))
      copies.append(pltpu.async_copy(
          stagea_v.at[j], outa_hbm.at[j].at[pl.ds(out0, CHUNK)], sem))
    for cp in copies:
      cp.wait()


@functools.cache
def _sc_gather():
  return functools.partial(
      pl.kernel,
      out_type=[
          jax.ShapeDtypeStruct((EMBED_DIM, BATCH, 8), jnp.float32),
          jax.ShapeDtypeStruct((EMBED_DIM, BATCH, 8), jnp.float32),
      ],
      mesh=plsc.VectorSubcoreMesh(core_axis_name="c", subcore_axis_name="s",
                                  num_cores=NUM_CORES,
                                  num_subcores=NUM_SUBCORES),
      scratch_types=(
          [pltpu.VMEM((NCHUNK, CHUNK), jnp.int32)] * 2
          + [pltpu.VMEM((CHUNK,), jnp.int32)] * (2 * EMBED_DIM)
          + [pltpu.VMEM((EMBED_DIM, CHUNK, 8), jnp.float32)] * 2
          + [pltpu.SemaphoreType.DMA]),
      compiler_params=pltpu.CompilerParams(use_tc_tiling_on_sc=False),
  )(_gather_body)


def _mlp_body(bufd_ref, bufa_ref, did_ref, aid_ref, w1d_ref, w1a_ref,
              b1_ref, w2_ref, b2_ref, o_ref):
  blk = did_ref.shape[0]
  oct8 = jax.lax.broadcasted_iota(jnp.int32, (blk, 8), 1)
  dm = (oct8 == (did_ref[...] & 7).reshape(blk, 1)).astype(jnp.float32)
  am = (oct8 == (aid_ref[...] & 7).reshape(blk, 1)).astype(jnp.float32)
  xd = jnp.concatenate(
      [jnp.sum(bufd_ref[j] * dm, axis=1, keepdims=True)
       for j in range(EMBED_DIM)], axis=1)
  xa = jnp.concatenate(
      [jnp.sum(bufa_ref[j] * am, axis=1, keepdims=True)
       for j in range(EMBED_DIM)], axis=1)
  h = jnp.dot(xd, w1d_ref[...], preferred_element_type=jnp.float32)
  h = h + jnp.dot(xa, w1a_ref[...], preferred_element_type=jnp.float32)
  h = jnp.maximum(h + b1_ref[...], 0.0)
  o_ref[...] = jnp.sum(h * w2_ref[...], axis=1) + b2_ref[0, 0]


def _mlp(bufd, bufa, did, aid, w1d, w1a, b1r, w2r, b2r):
  blk = 2048
  grid = BATCH // blk
  return pl.pallas_call(
      _mlp_body,
      out_shape=jax.ShapeDtypeStruct((BATCH,), jnp.float32),
      grid=(grid,),
      in_specs=[
          pl.BlockSpec((EMBED_DIM, blk, 8), lambda i: (0, i, 0)),
          pl.BlockSpec((EMBED_DIM, blk, 8), lambda i: (0, i, 0)),
          pl.BlockSpec((blk,), lambda i: (i,)),
          pl.BlockSpec((blk,), lambda i: (i,)),
          pl.BlockSpec((EMBED_DIM, HIDDEN), lambda i: (0, 0)),
          pl.BlockSpec((EMBED_DIM, HIDDEN), lambda i: (0, 0)),
          pl.BlockSpec((1, HIDDEN), lambda i: (0, 0)),
          pl.BlockSpec((1, HIDDEN), lambda i: (0, 0)),
          pl.BlockSpec((1, 1), lambda i: (0, 0)),
      ],
      out_specs=pl.BlockSpec((blk,), lambda i: (i,)),
  )(bufd, bufa, did, aid, w1d, w1a, b1r, w2r, b2r)


def kernel(domain_id, author_id, domain_table, author_table, W1, b1, W2, b2):
  did = domain_id.astype(jnp.int32)
  aid = author_id.astype(jnp.int32)
  dtab = _detile(domain_table.T, DV).reshape(-1, 8)
  atab = _detile(author_table.T, AV).reshape(-1, 8)
  did2 = did.reshape(ROW_BLOCKS, CHUNK)
  aid2 = aid.reshape(ROW_BLOCKS, CHUNK)
  bufd, bufa = _sc_gather()(did2, aid2, dtab, atab)
  w1d = W1[:EMBED_DIM]
  w1a = W1[EMBED_DIM:]
  b1r = b1.reshape(1, HIDDEN)
  w2r = W2.reshape(1, HIDDEN)
  b2r = b2.reshape(1, 1)
  return _mlp(bufd, bufa, did, aid, w1d, w1a, b1r, w2r, b2r)


# speedup vs baseline: 6.0849x; 1.0210x over previous
import functools

import jax
import jax.numpy as jnp
from jax import lax
from jax.experimental import pallas as pl
from jax.experimental.pallas import tpu as pltpu
from jax.experimental.pallas import tpu_sc as plsc

EMBED_DIM = 10
HIDDEN = 128
BATCH = 16384
DV = 100000
AV = 1000000

NUM_CORES = 2
NUM_SUBCORES = 16
NUM_WORKERS = NUM_CORES * NUM_SUBCORES
B_PER_W = BATCH // NUM_WORKERS
CHUNK = 128
NCHUNK = B_PER_W // CHUNK
ROW_BLOCKS = BATCH // CHUNK
LANES = 16

TILE_ROWS = 16
LANE_BLK = 65536


def _tr_body(tpb, src_ref, o_ref):
  x = src_ref[...]
  x16 = jnp.concatenate([x, x[:6]], axis=0)
  for t in range(tpb):
    o_ref[pl.ds(t * TILE_ROWS, TILE_ROWS), :] = x16[:, t * 128:(t + 1) * 128]


def _detile(tab_t, vocab):
  nb = -(-vocab // LANE_BLK)
  tpb = LANE_BLK // 128
  rows_pb = tpb * TILE_ROWS
  return pl.pallas_call(
      functools.partial(_tr_body, tpb),
      out_shape=jax.ShapeDtypeStruct((nb * rows_pb, 128), jnp.float32),
      grid=(nb,),
      in_specs=[pl.BlockSpec((EMBED_DIM, LANE_BLK), lambda c: (0, c))],
      out_specs=pl.BlockSpec((rows_pb, 128), lambda c: (c, 0)),
  )(tab_t)


def _gather_body(did_hbm, aid_hbm, dtab, atab, outd_hbm, outa_hbm, *scr):
  idxd_v, idxa_v = scr[0], scr[1]
  kds = scr[2:2 + EMBED_DIM]
  kas = scr[2 + EMBED_DIM:2 + 2 * EMBED_DIM]
  staged_v, stagea_v, sem = scr[-3], scr[-2], scr[-1]
  wid = lax.axis_index("s") * NUM_CORES + lax.axis_index("c")
  base = wid * NCHUNK
  pltpu.sync_copy(did_hbm.at[pl.ds(base, NCHUNK)], idxd_v)
  pltpu.sync_copy(aid_hbm.at[pl.ds(base, NCHUNK)], idxa_v)
  for c in range(NCHUNK):
    for g in range(CHUNK // LANES):
      sl = pl.ds(g * LANES, LANES)
      vd = idxd_v.at[c][sl]
      va = idxa_v.at[c][sl]
      bd = lax.shift_right_logical(vd, 3) + lax.shift_right_logical(vd, 7) * 240
      ba = lax.shift_right_logical(va, 3) + lax.shift_right_logical(va, 7) * 240
      for j in range(EMBED_DIM):
        kds[j][sl] = bd + j * 16
        kas[j][sl] = ba + j * 16
    copies = []
    for j in range(EMBED_DIM):
      copies.append(pltpu.async_copy(dtab.at[kds[j]], staged_v.at[j], sem))
      copies.append(pltpu.async_copy(atab.at[kas[j]], stagea_v.at[j], sem))
    for cp in copies:
      cp.wait()
    out0 = (base + c) * CHUNK
    copies = []
    for j in range(EMBED_DIM):
      copies.append(pltpu.async_copy(
          staged_v.at[j], outd_hbm.at[j].at[pl.ds(out0, CHUNK)], sem))
      copies.append(pltpu.async_copy(
          stagea_v.at[j], outa_hbm.at[j].at[pl.ds(out0, CHUNK)], sem))
    for cp in copies:
      cp.wait()


@functools.cache
def _sc_gather():
  return functools.partial(
      pl.kernel,
      out_type=[
          jax.ShapeDtypeStruct((EMBED_DIM, BATCH, 8), jnp.float32),
          jax.ShapeDtypeStruct((EMBED_DIM, BATCH, 8), jnp.float32),
      ],
      mesh=plsc.VectorSubcoreMesh(core_axis_name="c", subcore_axis_name="s",
                                  num_cores=NUM_CORES,
                                  num_subcores=NUM_SUBCORES),
      scratch_types=(
          [pltpu.VMEM((NCHUNK, CHUNK), jnp.int32)] * 2
          + [pltpu.VMEM((CHUNK,), jnp.int32)] * (2 * EMBED_DIM)
          + [pltpu.VMEM((EMBED_DIM, CHUNK, 8), jnp.float32)] * 2
          + [pltpu.SemaphoreType.DMA]),
      compiler_params=pltpu.CompilerParams(use_tc_tiling_on_sc=False),
  )(_gather_body)


def _mlp_body(bufd_ref, bufa_ref, did_ref, aid_ref, w1d_ref, w1a_ref,
              b1_ref, w2_ref, b2_ref, o_ref):
  blk = did_ref.shape[0]
  oct8 = jax.lax.broadcasted_iota(jnp.int32, (blk, 8), 1)
  dm = (oct8 == (did_ref[...] & 7).reshape(blk, 1)).astype(jnp.float32)
  am = (oct8 == (aid_ref[...] & 7).reshape(blk, 1)).astype(jnp.float32)
  xd = jnp.concatenate(
      [jnp.sum(bufd_ref[j] * dm, axis=1, keepdims=True)
       for j in range(EMBED_DIM)], axis=1)
  xa = jnp.concatenate(
      [jnp.sum(bufa_ref[j] * am, axis=1, keepdims=True)
       for j in range(EMBED_DIM)], axis=1)
  h = jnp.dot(xd, w1d_ref[...], preferred_element_type=jnp.float32)
  h = h + jnp.dot(xa, w1a_ref[...], preferred_element_type=jnp.float32)
  h = jnp.maximum(h + b1_ref[...], 0.0)
  o_ref[...] = jnp.sum(h * w2_ref[...], axis=1) + b2_ref[0, 0]


def _mlp(bufd, bufa, did, aid, w1d, w1a, b1r, w2r, b2r):
  blk = 2048
  grid = BATCH // blk
  return pl.pallas_call(
      _mlp_body,
      out_shape=jax.ShapeDtypeStruct((BATCH,), jnp.float32),
      grid=(grid,),
      in_specs=[
          pl.BlockSpec((EMBED_DIM, blk, 8), lambda i: (0, i, 0)),
          pl.BlockSpec((EMBED_DIM, blk, 8), lambda i: (0, i, 0)),
          pl.BlockSpec((blk,), lambda i: (i,)),
          pl.BlockSpec((blk,), lambda i: (i,)),
          pl.BlockSpec((EMBED_DIM, HIDDEN), lambda i: (0, 0)),
          pl.BlockSpec((EMBED_DIM, HIDDEN), lambda i: (0, 0)),
          pl.BlockSpec((1, HIDDEN), lambda i: (0, 0)),
          pl.BlockSpec((1, HIDDEN), lambda i: (0, 0)),
          pl.BlockSpec((1, 1), lambda i: (0, 0)),
      ],
      out_specs=pl.BlockSpec((blk,), lambda i: (i,)),
  )(bufd, bufa, did, aid, w1d, w1a, b1r, w2r, b2r)


def kernel(domain_id, author_id, domain_table, author_table, W1, b1, W2, b2):
  did = domain_id.astype(jnp.int32)
  aid = author_id.astype(jnp.int32)
  dtab = _detile(domain_table.T, DV).reshape(-1, 8)
  atab = _detile(author_table.T, AV).reshape(-1, 8)
  did2 = did.reshape(ROW_BLOCKS, CHUNK)
  aid2 = aid.reshape(ROW_BLOCKS, CHUNK)
  bufd, bufa = _sc_gather()(did2, aid2, dtab, atab)
  w1d = W1[:EMBED_DIM]
  w1a = W1[EMBED_DIM:]
  b1r = b1.reshape(1, HIDDEN)
  w2r = W2.reshape(1, HIDDEN)
  b2r = b2.reshape(1, 1)
  return _mlp(bufd, bufa, did, aid, w1d, w1a, b1r, w2r, b2r)
